# baseline (speedup 1.0000x reference)
import functools

import jax
import jax.numpy as jnp
from jax import lax
from jax.experimental import pallas as pl
from jax.experimental.pallas import tpu as pltpu
from jax.experimental.pallas import tpu_sc as plsc

_D = 64
_V = 1000000
_B = 4096 * 200

_info = plsc.get_sparse_core_info()
_NC, _NS = _info.num_cores, _info.num_subcores
_NW = _NC * _NS

_LT = 7813
_TPW = 245
_BLK = 2
_BC = _BLK * 128


_CMAX = (_V - _BC) // 128 * 128
_TAIL = _LT // _BLK * _BLK * 128


def _sc_detile(wt_hbm, wtail_hbm, out_hbm, stg, stg_t, tout, sem_in, sem_out):
    wid = lax.axis_index("s") * _NC + lax.axis_index("c")
    t0 = wid * _TPW
    nt = jnp.minimum(_TPW, jnp.maximum(_LT - t0, 0))
    nblk = (nt + _BLK - 1) // _BLK

    _iota = jax.lax.broadcasted_iota(jnp.int32, (16,), 0)

    def block(bi, _):
        c0 = (t0 + bi * _BLK) * 128
        c0 = jnp.minimum(c0, _CMAX)
        c0 = pl.multiple_of(c0, 128)
        pltpu.async_copy(wt_hbm.at[:, pl.ds(c0, _BC)], stg, sem_in).wait()

        @plsc.parallel_loop(0, _BC // 16, unroll=8)
        def _(cb):
            rowv = lax.div(cb * 16 + _iota, 2)
            colv0 = lax.rem(cb * 16 + _iota, 2) * 64
            for d in range(_D):
                v = stg[d, pl.ds(cb * 16, 16)]
                plsc.store_scatter(tout, [rowv, colv0 + d], v)

        co2 = pl.multiple_of(c0 // 2, 64)
        pltpu.async_copy(
            tout, out_hbm.at[pl.ds(co2, _BC // 2)], sem_out).wait()
        return ()

    lax.fori_loop(0, nblk, block, ())

    @pl.when(wid == 0)
    def _():
        pltpu.async_copy(wtail_hbm, stg_t, sem_in).wait()

        @plsc.parallel_loop(0, 4, unroll=1)
        def _(cb):
            rowv = lax.div(cb * 16 + _iota, 2)
            colv0 = lax.rem(cb * 16 + _iota, 2) * 64
            for d in range(_D):
                v = stg_t[d, pl.ds(cb * 16, 16)]
                plsc.store_scatter(tout, [rowv, colv0 + d], v)

        pltpu.async_copy(tout.at[pl.ds(0, 32)],
                         out_hbm.at[pl.ds(_TAIL // 2, 32)], sem_out).wait()


@jax.jit
def _compact(wt, wtail):
    mesh = plsc.VectorSubcoreMesh(core_axis_name="c", subcore_axis_name="s")
    k = functools.partial(
        pl.kernel,
        mesh=mesh,
        compiler_params=pltpu.CompilerParams(needs_layout_passes=False),
        out_type=jax.ShapeDtypeStruct((_V // 2, 128), jnp.float32),
        scratch_types=[
            pltpu.VMEM((_D, _BC), jnp.float32),
            pltpu.VMEM((_D, 64), jnp.float32),
            pltpu.VMEM((_BC // 2, 128), jnp.float32),
            pltpu.SemaphoreType.DMA,
            pltpu.SemaphoreType.DMA,
        ],
    )(_sc_detile)
    return k(wt, wtail)


_BPW = _B // _NW
_CHUNK = 256
_NCHUNK = _BPW // _CHUNK


def _sc_gather(idx_hbm, table_hbm, out_hbm,
               idxv0, idxv1, idx20, idx21,
               wide0, sel0, sel1,
               sem_idx, sem_g, semo0, semo1):
    wid = lax.axis_index("s") * _NC + lax.axis_index("c")
    base = wid * _BPW
    idx_v = (idxv0, idxv1)
    idx2_v = (idx20, idx21)
    wide_v = (wide0, wide0)
    sel_v = (sel0, sel1)
    sem_out = (semo0, semo1)

    def pair(g, _):
        for s in range(2):
            off = pl.multiple_of(base + (g * 2 + s) * _CHUNK, _CHUNK)
            pltpu.async_copy(idx_hbm.at[pl.ds(off, _CHUNK)],
                             idx_v[s].at[pl.ds(0, _CHUNK)], sem_idx).wait()
            for j in range(_CHUNK // 16):
                idx2_v[s][pl.ds(j * 16, 16)] = lax.shift_right_logical(
                    idx_v[s][pl.ds(j * 16, 16)], 1)
            cps = []
            for j in range(_CHUNK // 128):
                cps.append(pltpu.async_copy(
                    table_hbm.at[idx2_v[s].at[pl.ds(j * 128, 128)]],
                    wide_v[s].at[pl.ds(j * 128, 128)], sem_g))

            @pl.when(g > 0)
            def _():
                pltpu.make_async_copy(sel_v[s], out_hbm.at[pl.ds(off, _CHUNK)],
                                      sem_out[s]).wait()
            for cp in cps:
                cp.wait()

            @plsc.parallel_loop(0, _CHUNK, unroll=4)
            def _(i):
                b = (idx_v[s][pl.ds(i, 16)][0] & 1) * 64
                for j in range(4):
                    sel_v[s][i, pl.ds(j * 16, 16)] = (
                        wide_v[s][i, pl.ds(b + j * 16, 16)])

            pltpu.async_copy(sel_v[s], out_hbm.at[pl.ds(off, _CHUNK)],
                             sem_out[s])
        return ()

    lax.fori_loop(0, _NCHUNK // 2, pair, ())
    for s in range(2):
        off = pl.multiple_of(base + (_NCHUNK - 2 + s) * _CHUNK, _CHUNK)
        pltpu.make_async_copy(sel_v[s], out_hbm.at[pl.ds(off, _CHUNK)],
                              sem_out[s]).wait()


@jax.jit
def _embed(token_ids_flat, wpairs):
    mesh = plsc.VectorSubcoreMesh(core_axis_name="c", subcore_axis_name="s")
    k = functools.partial(
        pl.kernel,
        mesh=mesh,
        compiler_params=pltpu.CompilerParams(needs_layout_passes=False),
        out_type=jax.ShapeDtypeStruct((_B, _D), jnp.float32),
        scratch_types=[
            pltpu.VMEM((_CHUNK + 16,), jnp.int32),
            pltpu.VMEM((_CHUNK + 16,), jnp.int32),
            pltpu.VMEM((_CHUNK,), jnp.int32),
            pltpu.VMEM((_CHUNK,), jnp.int32),
            pltpu.VMEM((_CHUNK, 128), jnp.float32),
            pltpu.VMEM((_CHUNK, _D), jnp.float32),
            pltpu.VMEM((_CHUNK, _D), jnp.float32),
            pltpu.SemaphoreType.DMA,
            pltpu.SemaphoreType.DMA,
            pltpu.SemaphoreType.DMA,
            pltpu.SemaphoreType.DMA,
        ],
    )(_sc_gather)
    return k(token_ids_flat, wpairs)


def kernel(token_ids, weight):
    flat = token_ids.reshape(-1).astype(jnp.int32)
    wpairs = _compact(weight.T, weight[_TAIL:].T)
    out = _embed(flat, wpairs)
    return out.reshape(token_ids.shape + (weight.shape[1],))


# speedup vs baseline: 1.5837x; 1.5837x over previous
import functools

import jax
import jax.numpy as jnp
from jax import lax
from jax.experimental import pallas as pl
from jax.experimental.pallas import tpu as pltpu
from jax.experimental.pallas import tpu_sc as plsc

_D = 64
_V = 1000000
_B = 4096 * 200

_info = plsc.get_sparse_core_info()
_NC, _NS = _info.num_cores, _info.num_subcores
_NW = _NC * _NS
_BPW = _B // _NW
_CHUNK = 256
_NCHUNK = _BPW // _CHUNK


def _sc_gather(idx_hbm, table_hbm, out_hbm,
               idxv0, idxv1, idx20, idx21,
               wide0, sel0, sel1,
               sem_idx, sem_g, semo0, semo1):
    wid = lax.axis_index("s") * _NC + lax.axis_index("c")
    base = wid * _BPW
    idx_v = (idxv0, idxv1)
    idx2_v = (idx20, idx21)
    wide_v = (wide0, wide0)
    sel_v = (sel0, sel1)
    sem_out = (semo0, semo1)

    def pair(g, _):
        for s in range(2):
            off = pl.multiple_of(base + (g * 2 + s) * _CHUNK, _CHUNK)
            pltpu.async_copy(idx_hbm.at[pl.ds(off, _CHUNK)],
                             idx_v[s].at[pl.ds(0, _CHUNK)], sem_idx).wait()
            for j in range(_CHUNK // 16):
                idx2_v[s][pl.ds(j * 16, 16)] = lax.shift_right_logical(
                    idx_v[s][pl.ds(j * 16, 16)], 1)
            cps = []
            for j in range(_CHUNK // 128):
                cps.append(pltpu.async_copy(
                    table_hbm.at[idx2_v[s].at[pl.ds(j * 128, 128)]],
                    wide_v[s].at[pl.ds(j * 128, 128)], sem_g))

            @pl.when(g > 0)
            def _():
                pltpu.make_async_copy(sel_v[s], out_hbm.at[pl.ds(off, _CHUNK)],
                                      sem_out[s]).wait()
            for cp in cps:
                cp.wait()

            @plsc.parallel_loop(0, _CHUNK, unroll=8)
            def _(i):
                b = (idx_v[s][pl.ds(i, 16)][0] & 1) * 64
                for j in range(4):
                    sel_v[s][i, pl.ds(j * 16, 16)] = (
                        wide_v[s][i, pl.ds(b + j * 16, 16)])

            pltpu.async_copy(sel_v[s], out_hbm.at[pl.ds(off, _CHUNK)],
                             sem_out[s])
        return ()

    lax.fori_loop(0, _NCHUNK // 2, pair, ())
    for s in range(2):
        off = pl.multiple_of(base + (_NCHUNK - 2 + s) * _CHUNK, _CHUNK)
        pltpu.make_async_copy(sel_v[s], out_hbm.at[pl.ds(off, _CHUNK)],
                              sem_out[s]).wait()


@jax.jit
def _embed(token_ids_flat, wpairs):
    mesh = plsc.VectorSubcoreMesh(core_axis_name="c", subcore_axis_name="s")
    k = functools.partial(
        pl.kernel,
        mesh=mesh,
        compiler_params=pltpu.CompilerParams(needs_layout_passes=False),
        out_type=jax.ShapeDtypeStruct((_B, _D), jnp.float32),
        scratch_types=[
            pltpu.VMEM((_CHUNK + 16,), jnp.int32),
            pltpu.VMEM((_CHUNK + 16,), jnp.int32),
            pltpu.VMEM((_CHUNK,), jnp.int32),
            pltpu.VMEM((_CHUNK,), jnp.int32),
            pltpu.VMEM((_CHUNK, 128), jnp.float32),
            pltpu.VMEM((_CHUNK, _D), jnp.float32),
            pltpu.VMEM((_CHUNK, _D), jnp.float32),
            pltpu.SemaphoreType.DMA,
            pltpu.SemaphoreType.DMA,
            pltpu.SemaphoreType.DMA,
            pltpu.SemaphoreType.DMA,
        ],
    )(_sc_gather)
    return k(token_ids_flat, wpairs)


def kernel(token_ids, weight):
    flat = token_ids.reshape(-1).astype(jnp.int32)
    w128 = weight.reshape(_V // 2, 128)
    out = _embed(flat, w128)
    return out.reshape(token_ids.shape + (weight.shape[1],))


# speedup vs baseline: 1.6489x; 1.0412x over previous
import functools

import jax
import jax.numpy as jnp
from jax import lax
from jax.experimental import pallas as pl
from jax.experimental.pallas import tpu as pltpu
from jax.experimental.pallas import tpu_sc as plsc

_D = 64
_V = 1000000
_B = 4096 * 200

_info = plsc.get_sparse_core_info()
_NC, _NS = _info.num_cores, _info.num_subcores
_NW = _NC * _NS
_BPW = _B // _NW
_CHUNK = 256
_NCHUNK = _BPW // _CHUNK


def _sc_gather(idx_hbm, table_hbm, out_hbm,
               idxv0, idxv1, idx20, idx21,
               wide0, sel0, sel1,
               sem_idx, sem_g, semo0, semo1):
    wid = lax.axis_index("s") * _NC + lax.axis_index("c")
    base = wid * _BPW
    idx_v = (idxv0, idxv1)
    idx2_v = (idx20, idx21)
    wide_v = (wide0, wide0)
    sel_v = (sel0, sel1)
    sem_out = (semo0, semo1)

    pltpu.async_copy(idx_hbm.at[pl.ds(pl.multiple_of(base, _CHUNK), _CHUNK)],
                     idx_v[0].at[pl.ds(0, _CHUNK)], sem_idx)

    def pair(g, _):
        for s in range(2):
            i = g * 2 + s
            off = pl.multiple_of(base + i * _CHUNK, _CHUNK)
            pltpu.make_async_copy(idx_hbm.at[pl.ds(off, _CHUNK)],
                                  idx_v[s].at[pl.ds(0, _CHUNK)],
                                  sem_idx).wait()

            @pl.when(i + 1 < _NCHUNK)
            def _():
                offn = pl.multiple_of(base + (i + 1) * _CHUNK, _CHUNK)
                pltpu.async_copy(idx_hbm.at[pl.ds(offn, _CHUNK)],
                                 idx_v[1 - s].at[pl.ds(0, _CHUNK)], sem_idx)

            for j in range(_CHUNK // 16):
                idx2_v[s][pl.ds(j * 16, 16)] = lax.shift_right_logical(
                    idx_v[s][pl.ds(j * 16, 16)], 1)
            cps = []
            for j in range(_CHUNK // 128):
                cps.append(pltpu.async_copy(
                    table_hbm.at[idx2_v[s].at[pl.ds(j * 128, 128)]],
                    wide_v[s].at[pl.ds(j * 128, 128)], sem_g))

            @pl.when(g > 0)
            def _():
                pltpu.make_async_copy(sel_v[s], out_hbm.at[pl.ds(off, _CHUNK)],
                                      sem_out[s]).wait()
            for cp in cps:
                cp.wait()

            @plsc.parallel_loop(0, _CHUNK, unroll=8)
            def _(i):
                b = (idx_v[s][pl.ds(i, 16)][0] & 1) * 64
                for j in range(4):
                    sel_v[s][i, pl.ds(j * 16, 16)] = (
                        wide_v[s][i, pl.ds(b + j * 16, 16)])

            pltpu.async_copy(sel_v[s], out_hbm.at[pl.ds(off, _CHUNK)],
                             sem_out[s])
        return ()

    lax.fori_loop(0, _NCHUNK // 2, pair, ())
    for s in range(2):
        off = pl.multiple_of(base + (_NCHUNK - 2 + s) * _CHUNK, _CHUNK)
        pltpu.make_async_copy(sel_v[s], out_hbm.at[pl.ds(off, _CHUNK)],
                              sem_out[s]).wait()


@jax.jit
def _embed(token_ids_flat, wpairs):
    mesh = plsc.VectorSubcoreMesh(core_axis_name="c", subcore_axis_name="s")
    k = functools.partial(
        pl.kernel,
        mesh=mesh,
        compiler_params=pltpu.CompilerParams(needs_layout_passes=False),
        out_type=jax.ShapeDtypeStruct((_B, _D), jnp.float32),
        scratch_types=[
            pltpu.VMEM((_CHUNK + 16,), jnp.int32),
            pltpu.VMEM((_CHUNK + 16,), jnp.int32),
            pltpu.VMEM((_CHUNK,), jnp.int32),
            pltpu.VMEM((_CHUNK,), jnp.int32),
            pltpu.VMEM((_CHUNK, 128), jnp.float32),
            pltpu.VMEM((_CHUNK, _D), jnp.float32),
            pltpu.VMEM((_CHUNK, _D), jnp.float32),
            pltpu.SemaphoreType.DMA,
            pltpu.SemaphoreType.DMA,
            pltpu.SemaphoreType.DMA,
            pltpu.SemaphoreType.DMA,
        ],
    )(_sc_gather)
    return k(token_ids_flat, wpairs)


def kernel(token_ids, weight):
    flat = token_ids.reshape(-1).astype(jnp.int32)
    w128 = weight.reshape(_V // 2, 128)
    out = _embed(flat, w128)
    return out.reshape(token_ids.shape + (weight.shape[1],))


# speedup vs baseline: 1.7208x; 1.0436x over previous
import functools

import jax
import jax.numpy as jnp
from jax import lax
from jax.experimental import pallas as pl
from jax.experimental.pallas import tpu as pltpu
from jax.experimental.pallas import tpu_sc as plsc

_D = 64
_V = 1000000
_B = 4096 * 200

_info = plsc.get_sparse_core_info()
_NC, _NS = _info.num_cores, _info.num_subcores
_NW = _NC * _NS
_BPW = _B // _NW
_CHUNK = 160
_NCHUNK = _BPW // _CHUNK
_GW = _CHUNK // 2


def _sc_gather(idx_hbm, table_hbm, out_hbm,
               idxv0, idxv1, idx20, idx21, par0, par1,
               wide0, wide1, sel0, sel1,
               sem_idx, semg0, semg1, semo0, semo1):
    wid = lax.axis_index("s") * _NC + lax.axis_index("c")
    base = wid * _BPW
    idx_v = (idxv0, idxv1)
    idx2_v = (idx20, idx21)
    par_v = (par0, par1)
    wide_v = (wide0, wide1)
    sel_v = (sel0, sel1)
    sem_g = (semg0, semg1)
    sem_out = (semo0, semo1)

    def stage(i, s):
        off = pl.multiple_of(base + i * _CHUNK, _CHUNK)
        pltpu.make_async_copy(idx_hbm.at[pl.ds(off, _CHUNK)],
                              idx_v[s].at[pl.ds(0, _CHUNK)], sem_idx).wait()

        @pl.when(i + 1 < _NCHUNK)
        def _():
            offn = pl.multiple_of(base + (i + 1) * _CHUNK, _CHUNK)
            pltpu.async_copy(idx_hbm.at[pl.ds(offn, _CHUNK)],
                             idx_v[1 - s].at[pl.ds(0, _CHUNK)], sem_idx)

        for j in range(_CHUNK // 16):
            t = idx_v[s][pl.ds(j * 16, 16)]
            idx2_v[s][pl.ds(j * 16, 16)] = lax.shift_right_logical(t, 1)
            par_v[s][pl.ds(j * 16, 16)] = (t & 1) * 64
        for j in range(_CHUNK // _GW):
            pltpu.async_copy(
                table_hbm.at[idx2_v[s].at[pl.ds(j * _GW, _GW)]],
                wide_v[s].at[pl.ds(j * _GW, _GW)], sem_g[s])

    def complete(i, s, drain):
        off = pl.multiple_of(base + i * _CHUNK, _CHUNK)
        for j in range(_CHUNK // _GW):
            pltpu.make_async_copy(
                table_hbm.at[idx2_v[s].at[pl.ds(j * _GW, _GW)]],
                wide_v[s].at[pl.ds(j * _GW, _GW)], sem_g[s]).wait()
        if drain:
            @pl.when(i >= 2)
            def _():
                pltpu.make_async_copy(sel_v[s],
                                      out_hbm.at[pl.ds(off, _CHUNK)],
                                      sem_out[s]).wait()

        @plsc.parallel_loop(0, _CHUNK, unroll=8)
        def _(r):
            b = par_v[s][pl.ds(r, 16)][0]
            for j in range(4):
                sel_v[s][r, pl.ds(j * 16, 16)] = (
                    wide_v[s][r, pl.ds(b + j * 16, 16)])

        pltpu.async_copy(sel_v[s], out_hbm.at[pl.ds(off, _CHUNK)],
                         sem_out[s])

    pltpu.async_copy(idx_hbm.at[pl.ds(pl.multiple_of(base, _CHUNK), _CHUNK)],
                     idx_v[0].at[pl.ds(0, _CHUNK)], sem_idx)
    stage(0, 0)

    def pair(g, _):
        i = g * 2
        stage(i + 1, 1)
        complete(i, 0, True)

        @pl.when(i + 2 < _NCHUNK)
        def _():
            stage(i + 2, 0)
        complete(i + 1, 1, True)
        return ()

    lax.fori_loop(0, _NCHUNK // 2, pair, ())
    for s in range(2):
        off = pl.multiple_of(base + (_NCHUNK - 2 + s) * _CHUNK, _CHUNK)
        pltpu.make_async_copy(sel_v[s], out_hbm.at[pl.ds(off, _CHUNK)],
                              sem_out[s]).wait()


@jax.jit
def _embed(token_ids_flat, wpairs):
    mesh = plsc.VectorSubcoreMesh(core_axis_name="c", subcore_axis_name="s")
    k = functools.partial(
        pl.kernel,
        mesh=mesh,
        compiler_params=pltpu.CompilerParams(needs_layout_passes=False),
        out_type=jax.ShapeDtypeStruct((_B, _D), jnp.float32),
        scratch_types=[
            pltpu.VMEM((_CHUNK + 16,), jnp.int32),
            pltpu.VMEM((_CHUNK + 16,), jnp.int32),
            pltpu.VMEM((_CHUNK,), jnp.int32),
            pltpu.VMEM((_CHUNK,), jnp.int32),
            pltpu.VMEM((_CHUNK + 16,), jnp.int32),
            pltpu.VMEM((_CHUNK + 16,), jnp.int32),
            pltpu.VMEM((_CHUNK, 128), jnp.float32),
            pltpu.VMEM((_CHUNK, 128), jnp.float32),
            pltpu.VMEM((_CHUNK, _D), jnp.float32),
            pltpu.VMEM((_CHUNK, _D), jnp.float32),
            pltpu.SemaphoreType.DMA,
            pltpu.SemaphoreType.DMA,
            pltpu.SemaphoreType.DMA,
            pltpu.SemaphoreType.DMA,
            pltpu.SemaphoreType.DMA,
        ],
    )(_sc_gather)
    return k(token_ids_flat, wpairs)


def kernel(token_ids, weight):
    flat = token_ids.reshape(-1).astype(jnp.int32)
    w128 = weight.reshape(_V // 2, 128)
    out = _embed(flat, w128)
    return out.reshape(token_ids.shape + (weight.shape[1],))
